# baseline (speedup 1.0000x reference)
import functools

import jax
import jax.numpy as jnp
import numpy as np
from jax.experimental import pallas as pl
from jax.experimental.pallas import tpu as pltpu

_PREC = jax.lax.Precision.HIGHEST


def _dot(a, b):
    return jax.lax.dot_general(a, b, (((a.ndim - 1,), (0,)), ((), ())),
                               precision=_PREC)


N_PTS = 8192
F_LOC = 6
NUM_CLASSES = 13
MAX_NBR = 64


def _mlp2d(layers, h):
    for (W, b, g, beta) in layers:
        h = jax.nn.relu(h @ W + b)
        mu = jnp.mean(h, axis=0)
        var = jnp.var(h, axis=0)
        h = (h - mu) / jnp.sqrt(var + 1e-5) * g + beta
    return h


def _mlp3d_masked(layers, h, mask):
    m = mask[..., None].astype(h.dtype)
    cnt = jnp.maximum(jnp.sum(m), 1.0)
    for (W, b, g, beta) in layers:
        h = jax.nn.relu(h @ W + b)
        mu = jnp.sum(h * m, axis=(0, 1)) / cnt
        var = jnp.sum(((h - mu) ** 2) * m, axis=(0, 1)) / cnt
        h = (h - mu) / jnp.sqrt(var + 1e-5) * g + beta
    return h


def _fps_kernel(p_ref, out_ref, dmin_ref, *, M, N, L):
    lin = (jax.lax.broadcasted_iota(jnp.int32, (8, L), 0) * L
           + jax.lax.broadcasted_iota(jnp.int32, (8, L), 1))
    valid = lin < N
    X = p_ref[0]
    Y = p_ref[1]
    Z = p_ref[2]
    dmin_ref[...] = jnp.where(valid, jnp.inf, -jnp.inf)
    out_ref[0] = jnp.int32(0)

    def body(i, last):
        m = (lin == last).astype(jnp.float32)
        px = jnp.sum(X * m)
        py = jnp.sum(Y * m)
        pz = jnp.sum(Z * m)
        d = (X - px) ** 2 + (Y - py) ** 2 + (Z - pz) ** 2
        dm = jnp.minimum(dmin_ref[...], d)
        dmin_ref[...] = dm
        mx = jnp.max(dm)
        nxt = jnp.min(jnp.where(dm == mx, lin, N))
        out_ref[i] = nxt
        return nxt

    jax.lax.fori_loop(1, M, body, jnp.int32(0))


def _fps(pos, ratio):
    N = pos.shape[0]
    M = max(int(N * ratio), 1)
    L = int(np.ceil(N / 8 / 128)) * 128
    Mpad = int(np.ceil(M / 8)) * 8
    p = jnp.pad(pos, ((0, 8 * L - N), (0, 0))).T.reshape(3, 8, L)
    idx = pl.pallas_call(
        functools.partial(_fps_kernel, M=M, N=N, L=L),
        in_specs=[pl.BlockSpec((3, 8, L), lambda: (0, 0, 0))],
        out_specs=pl.BlockSpec(memory_space=pltpu.SMEM),
        out_shape=jax.ShapeDtypeStruct((Mpad,), jnp.int32),
        scratch_shapes=[pltpu.VMEM((8, L), jnp.float32)],
    )(p)
    return idx[:M]


def _cumsum_lanes(x):
    L = x.shape[1]
    liota = jax.lax.broadcasted_iota(jnp.int32, x.shape, 1)
    s = 1
    while s < L:
        shifted = pltpu.roll(x, s, 1)
        x = x + jnp.where(liota >= s, shifted, 0)
        s *= 2
    return x


def _sa_select_kernel(pq_ref, ps_ref, xp_ref, w1_ref, b1_ref,
                      h1_ref, mask_ref, s_ref, ss_ref, cnt_ref,
                      *, M, N, K, C1, Cx, r2, tileM):
    i = pl.program_id(0)
    q = pq_ref[...]
    qx = q[:, 0:1]
    qy = q[:, 1:2]
    qz = q[:, 2:3]
    sx = ps_ref[0:1, :]
    sy = ps_ref[1:2, :]
    sz = ps_ref[2:3, :]
    d2 = (qx - sx) ** 2 + (qy - sy) ** 2 + (qz - sz) ** 2
    d2i = jax.lax.bitcast_convert_type(d2, jnp.int32)
    r2i = jax.lax.bitcast_convert_type(jnp.float32(r2), jnp.int32)
    rowv = (i * tileM + jax.lax.broadcasted_iota(jnp.int32, (tileM, 1), 0)) < M
    c_r = jnp.sum((d2i <= r2i).astype(jnp.int32), axis=1, keepdims=True)
    target = jnp.minimum(c_r, K)

    def bis(_, state):
        lo, hi = state
        mid = (lo + hi) >> 1
        cm = jnp.sum((d2i <= mid).astype(jnp.int32), axis=1, keepdims=True)
        ge = cm >= target
        return (jnp.where(ge, lo, mid), jnp.where(ge, mid, hi))

    lo0 = jnp.full_like(target, -1)
    hi0 = jnp.full_like(target, r2i)
    _, t = jax.lax.fori_loop(0, 31, bis, (lo0, hi0))
    below = d2i < t
    boundary = d2i == t
    nbelow = jnp.sum(below.astype(jnp.int32), axis=1, keepdims=True)
    bnd_i = boundary.astype(jnp.int32)
    b_rank = _cumsum_lanes(bnd_i) - bnd_i
    sel = below | (boundary & (b_rank < (target - nbelow)))
    seli = sel.astype(jnp.int32)
    rank = _cumsum_lanes(seli) - seli
    rankc = jnp.where(sel, rank, -1)

    kiota = jax.lax.broadcasted_iota(jnp.int32, (tileM, K), 1)
    maskf = ((kiota < target) & rowv).astype(jnp.float32)
    mask_ref[...] = maskf

    w1 = w1_ref[...]
    b1 = b1_ref[...]
    xp = xp_ref[...]
    s = jnp.zeros((1, C1), jnp.float32)
    ss = jnp.zeros((1, C1), jnp.float32)
    for k in range(K):
        oh = (rankc == k).astype(jnp.float32)
        feat = _dot(oh, xp)
        msg = jnp.concatenate([feat[:, :Cx], feat[:, Cx:Cx + 3] - q], axis=1)
        hk = jnp.maximum(msg @ w1 + b1, 0.0)
        mk = maskf[:, k:k + 1]
        hkm = hk * mk
        s = s + jnp.sum(hkm, axis=0, keepdims=True)
        ss = ss + jnp.sum(hkm * hk, axis=0, keepdims=True)
        h1_ref[:, k * C1:(k + 1) * C1] = hk

    @pl.when(i == 0)
    def _():
        s_ref[...] = jnp.zeros_like(s_ref)
        ss_ref[...] = jnp.zeros_like(ss_ref)
        cnt_ref[...] = jnp.zeros_like(cnt_ref)

    s_ref[...] += s
    ss_ref[...] += ss
    cnt_ref[...] += jnp.sum(maskf)


def _sa_mid_kernel(h_ref, mask_ref, s_ref, ss_ref, cnt_ref,
                   g_ref, be_ref, w_ref, b_ref,
                   ho_ref, so_ref, sso_ref, *, K, Cin, Cout):
    i = pl.program_id(0)
    cnt = jnp.maximum(cnt_ref[0, 0], 1.0)
    mu = s_ref[...] / cnt
    var = ss_ref[...] / cnt - mu * mu
    scale = g_ref[...] * jax.lax.rsqrt(var + 1e-5)
    shift = be_ref[...] - mu * scale
    w = w_ref[...]
    b = b_ref[...]
    s = jnp.zeros((1, Cout), jnp.float32)
    ss = jnp.zeros((1, Cout), jnp.float32)
    for k in range(K):
        hn = h_ref[:, k * Cin:(k + 1) * Cin] * scale + shift
        hk = jnp.maximum(hn @ w + b, 0.0)
        mk = mask_ref[:, k:k + 1]
        hkm = hk * mk
        s = s + jnp.sum(hkm, axis=0, keepdims=True)
        ss = ss + jnp.sum(hkm * hk, axis=0, keepdims=True)
        ho_ref[:, k * Cout:(k + 1) * Cout] = hk

    @pl.when(i == 0)
    def _():
        so_ref[...] = jnp.zeros_like(so_ref)
        sso_ref[...] = jnp.zeros_like(sso_ref)

    so_ref[...] += s
    sso_ref[...] += ss


def _sa_final_kernel(h_ref, mask_ref, s_ref, ss_ref, cnt_ref, g_ref, be_ref,
                     out_ref, *, K, Cin):
    cnt = jnp.maximum(cnt_ref[0, 0], 1.0)
    mu = s_ref[...] / cnt
    var = ss_ref[...] / cnt - mu * mu
    scale = g_ref[...] * jax.lax.rsqrt(var + 1e-5)
    shift = be_ref[...] - mu * scale
    acc = jnp.full(out_ref.shape, -jnp.inf, jnp.float32)
    for k in range(K):
        hn = h_ref[:, k * Cin:(k + 1) * Cin] * scale + shift
        mk = mask_ref[:, k:k + 1] > 0.0
        acc = jnp.maximum(acc, jnp.where(mk, hn, -jnp.inf))
    out_ref[...] = acc


def _sa_module(layers, x, pos, ratio, r):
    idx = _fps(pos, ratio)
    pos_q = pos[idx]
    N = pos.shape[0]
    M = pos_q.shape[0]
    K = MAX_NBR
    Cx = x.shape[1]
    tileM = 128
    Mpad = int(np.ceil(M / tileM)) * tileM
    Npad = int(np.ceil(N / 128)) * 128
    (W1, b1, g1, be1), (W2, b2, g2, be2), (W3, b3, g3, be3) = layers
    C1, C2, C3 = W1.shape[1], W2.shape[1], W3.shape[1]

    pq = jnp.pad(pos_q, ((0, Mpad - M), (0, 0)))
    psT = jnp.pad(pos, ((0, Npad - N), (0, 0)), constant_values=1e4).T
    xp = jnp.pad(jnp.concatenate([x, pos], axis=1), ((0, Npad - N), (0, 0)))

    grid = (Mpad // tileM,)
    rep = lambda shape: pl.BlockSpec(shape, lambda i: (0,) * len(shape))
    h1, maskf, s1, ss1, cnt = pl.pallas_call(
        functools.partial(_sa_select_kernel, M=M, N=N, K=K, C1=C1, Cx=Cx,
                          r2=r * r, tileM=tileM),
        grid=grid,
        in_specs=[
            pl.BlockSpec((tileM, 3), lambda i: (i, 0)),
            rep((3, Npad)),
            rep((Npad, Cx + 3)),
            rep((Cx + 3, C1)),
            rep((1, C1)),
        ],
        out_specs=[
            pl.BlockSpec((tileM, K * C1), lambda i: (i, 0)),
            pl.BlockSpec((tileM, K), lambda i: (i, 0)),
            rep((1, C1)),
            rep((1, C1)),
            rep((1, 1)),
        ],
        out_shape=[
            jax.ShapeDtypeStruct((Mpad, K * C1), jnp.float32),
            jax.ShapeDtypeStruct((Mpad, K), jnp.float32),
            jax.ShapeDtypeStruct((1, C1), jnp.float32),
            jax.ShapeDtypeStruct((1, C1), jnp.float32),
            jax.ShapeDtypeStruct((1, 1), jnp.float32),
        ],
    )(pq, psT, xp, W1, b1.reshape(1, -1))

    def mid(h, s, ss, g, be, W, b, Cin, Cout):
        return pl.pallas_call(
            functools.partial(_sa_mid_kernel, K=K, Cin=Cin, Cout=Cout),
            grid=grid,
            in_specs=[
                pl.BlockSpec((tileM, K * Cin), lambda i: (i, 0)),
                pl.BlockSpec((tileM, K), lambda i: (i, 0)),
                rep((1, Cin)), rep((1, Cin)), rep((1, 1)),
                rep((1, Cin)), rep((1, Cin)),
                rep((Cin, Cout)), rep((1, Cout)),
            ],
            out_specs=[
                pl.BlockSpec((tileM, K * Cout), lambda i: (i, 0)),
                rep((1, Cout)),
                rep((1, Cout)),
            ],
            out_shape=[
                jax.ShapeDtypeStruct((Mpad, K * Cout), jnp.float32),
                jax.ShapeDtypeStruct((1, Cout), jnp.float32),
                jax.ShapeDtypeStruct((1, Cout), jnp.float32),
            ],
        )(h, maskf, s, ss, cnt, g.reshape(1, -1), be.reshape(1, -1),
          W, b.reshape(1, -1))

    h2, s2, ss2 = mid(h1, s1, ss1, g1, be1, W2, b2, C1, C2)
    h3, s3, ss3 = mid(h2, s2, ss2, g2, be2, W3, b3, C2, C3)

    out = pl.pallas_call(
        functools.partial(_sa_final_kernel, K=K, Cin=C3),
        grid=grid,
        in_specs=[
            pl.BlockSpec((tileM, K * C3), lambda i: (i, 0)),
            pl.BlockSpec((tileM, K), lambda i: (i, 0)),
            rep((1, C3)), rep((1, C3)), rep((1, 1)),
            rep((1, C3)), rep((1, C3)),
        ],
        out_specs=pl.BlockSpec((tileM, C3), lambda i: (i, 0)),
        out_shape=jax.ShapeDtypeStruct((Mpad, C3), jnp.float32),
    )(h3, maskf, s3, ss3, cnt, g3.reshape(1, -1), be3.reshape(1, -1))
    return out[:M], pos_q


def _interp_kernel(pd_ref, ps_ref, xs_ref, out_ref, *, k):
    q = pd_ref[...]
    qx = q[:, 0:1]
    qy = q[:, 1:2]
    qz = q[:, 2:3]
    sx = ps_ref[0:1, :]
    sy = ps_ref[1:2, :]
    sz = ps_ref[2:3, :]
    d2 = (qx - sx) ** 2 + (qy - sy) ** 2 + (qz - sz) ** 2
    W = jnp.zeros_like(d2)
    wsum = jnp.zeros_like(qx)
    for _ in range(k):
        m = jnp.min(d2, axis=1, keepdims=True)
        oh = d2 == m
        wk = 1.0 / jnp.maximum(m, 1e-16)
        W = W + jnp.where(oh, wk, 0.0)
        wsum = wsum + wk
        d2 = jnp.where(oh, jnp.inf, d2)
    out_ref[...] = _dot(W, xs_ref[...]) / wsum


def _knn_interp(x_src, pos_src, pos_dst, k):
    S, C = x_src.shape
    D = pos_dst.shape[0]
    Spad = int(np.ceil(S / 128)) * 128
    TILE = 512
    Dpad = int(np.ceil(D / TILE)) * TILE
    psT = jnp.pad(pos_src, ((0, Spad - S), (0, 0)), constant_values=1e4).T
    xs = jnp.pad(x_src, ((0, Spad - S), (0, 0)))
    pd = jnp.pad(pos_dst, ((0, Dpad - D), (0, 0)))
    out = pl.pallas_call(
        functools.partial(_interp_kernel, k=k),
        grid=(Dpad // TILE,),
        in_specs=[
            pl.BlockSpec((TILE, 3), lambda i: (i, 0)),
            pl.BlockSpec((3, Spad), lambda i: (0, 0)),
            pl.BlockSpec((Spad, C), lambda i: (0, 0)),
        ],
        out_specs=pl.BlockSpec((TILE, C), lambda i: (i, 0)),
        out_shape=jax.ShapeDtypeStruct((Dpad, C), jnp.float32),
    )(pd, psT, xs)
    return out[:D]


def _bn_stack(h, layer_refs, rv, M):
    for (w, b, g, be) in layer_refs:
        h = jnp.maximum(h @ w[...] + b[...], 0.0)
        hm = h * rv
        mu = jnp.sum(hm, axis=0, keepdims=True) / M
        var = jnp.sum(hm * h, axis=0, keepdims=True) / M - mu * mu
        sc = g[...] * jax.lax.rsqrt(var + 1e-5)
        h = h * sc + (be[...] - mu * sc)
    return h


def _mid_kernel(x2_ref, p2_ref, *rest, M):
    wr = rest[:20]
    out_ref = rest[20]
    sa3 = [wr[0:4], wr[4:8], wr[8:12]]
    fp3 = [wr[12:16], wr[16:20]]
    Mpad = x2_ref.shape[0]
    rv = (jax.lax.broadcasted_iota(jnp.int32, (Mpad, 1), 0) < M).astype(jnp.float32)
    x2 = x2_ref[...]
    h = jnp.concatenate([x2, p2_ref[...]], axis=1)
    h = _bn_stack(h, sa3, rv, M)
    x3 = jnp.max(jnp.where(rv > 0.0, h, -jnp.inf), axis=0, keepdims=True)
    f = jnp.concatenate([jnp.broadcast_to(x3, (Mpad, x3.shape[1])), x2], axis=1)
    out_ref[...] = _bn_stack(f, fp3, rv, M)


def _mid(x2, pos2, params):
    M = x2.shape[0]
    Mpad = int(np.ceil(M / 8)) * 8
    x2p = jnp.pad(x2, ((0, Mpad - M), (0, 0)))
    p2p = jnp.pad(pos2, ((0, Mpad - M), (0, 0)))
    wargs = []
    for (W, b, g, be) in list(params['sa3']) + list(params['fp3']):
        wargs += [W, b.reshape(1, -1), g.reshape(1, -1), be.reshape(1, -1)]
    out = pl.pallas_call(
        functools.partial(_mid_kernel, M=M),
        out_shape=jax.ShapeDtypeStruct((Mpad, 256), jnp.float32),
    )(x2p, p2p, *wargs)
    return out[:M]


def _fp2_kernel(f_ref, x1_ref, *rest, M):
    wr = rest[:8]
    out_ref = rest[8]
    fp2 = [wr[0:4], wr[4:8]]
    Mpad = f_ref.shape[0]
    rv = (jax.lax.broadcasted_iota(jnp.int32, (Mpad, 1), 0) < M).astype(jnp.float32)
    h = jnp.concatenate([f_ref[...], x1_ref[...]], axis=1)
    out_ref[...] = _bn_stack(h, fp2, rv, M)


def _fp2(f, x1, params):
    M = f.shape[0]
    Mpad = int(np.ceil(M / 8)) * 8
    fp = jnp.pad(f, ((0, Mpad - M), (0, 0)))
    x1p = jnp.pad(x1, ((0, Mpad - M), (0, 0)))
    wargs = []
    for (W, b, g, be) in params['fp2']:
        wargs += [W, b.reshape(1, -1), g.reshape(1, -1), be.reshape(1, -1)]
    out = pl.pallas_call(
        functools.partial(_fp2_kernel, M=M),
        out_shape=jax.ShapeDtypeStruct((Mpad, 128), jnp.float32),
    )(fp, x1p, *wargs)
    return out[:M]


def _fp1_head_kernel(f_ref, x_ref, *rest, M):
    wr = rest[:12]
    lr = rest[12:18]
    out_ref = rest[18]
    fp1 = [wr[0:4], wr[4:8], wr[8:12]]
    Mpad = f_ref.shape[0]
    rv = (jax.lax.broadcasted_iota(jnp.int32, (Mpad, 1), 0) < M).astype(jnp.float32)
    h = jnp.concatenate([f_ref[...], x_ref[...]], axis=1)
    h = _bn_stack(h, fp1, rv, M)
    h = jnp.maximum(h @ lr[0][...] + lr[1][...], 0.0)
    h = h @ lr[2][...] + lr[3][...]
    h = h @ lr[4][...] + lr[5][...]
    out_ref[...] = jax.nn.log_softmax(h, axis=-1)


def _fp1_head(f, x, params):
    M = f.shape[0]
    wargs = []
    for (W, b, g, be) in params['fp1']:
        wargs += [W, b.reshape(1, -1), g.reshape(1, -1), be.reshape(1, -1)]
    for nm in ('lin1', 'lin2', 'lin3'):
        W, b = params[nm]
        wargs += [W, b.reshape(1, -1)]
    return pl.pallas_call(
        functools.partial(_fp1_head_kernel, M=M),
        out_shape=jax.ShapeDtypeStruct((M, NUM_CLASSES), jnp.float32),
    )(f, x, *wargs)


def kernel(x, pos, batch, params):
    x1, pos1 = _sa_module(params['sa1'], x, pos, 0.2, 0.2)
    x2, pos2 = _sa_module(params['sa2'], x1, pos1, 0.25, 0.4)
    f = _mid(x2, pos2, params)
    f = _knn_interp(f, pos2, pos1, 3)
    f = _fp2(f, x1, params)
    f = _knn_interp(f, pos1, pos, 3)
    return _fp1_head(f, x, params)


def _radius_gather_x(pos, pos_q, r, K):
    d2 = jnp.sum((pos_q[:, None, :] - pos[None, :, :]) ** 2, axis=-1)
    score = jnp.where(d2 <= r * r, -d2, -jnp.inf)
    vals, nbr = jax.lax.top_k(score, K)
    mask = vals > -jnp.inf
    nbr = jnp.where(mask, nbr, 0)
    return nbr, mask


def _sa_module_xla(layers, x, pos, ratio, r):
    idx = _fps(pos, ratio)
    pos_q = pos[idx]
    nbr, mask = _radius_gather_x(pos, pos_q, r, MAX_NBR)
    x_j = x[nbr]
    rel = pos[nbr] - pos_q[:, None, :]
    msg = jnp.concatenate([x_j, rel], axis=-1)
    h = _mlp3d_masked(layers, msg, mask)
    h = jnp.where(mask[:, :, None], h, -jnp.inf)
    out = jnp.max(h, axis=1)
    return out, pos_q


def kernel(x, pos, batch, params):
    x1, pos1 = _sa_module_xla(params['sa1'], x, pos, 0.2, 0.2)
    x2, pos2 = _sa_module_xla(params['sa2'], x1, pos1, 0.25, 0.4)
    f = _mid(x2, pos2, params)
    f = _knn_interp(f, pos2, pos1, 3)
    f = _fp2(f, x1, params)
    f = _knn_interp(f, pos1, pos, 3)
    return _fp1_head(f, x, params)
